# baseline (speedup 1.0000x reference)
import functools

import jax
import jax.numpy as jnp
from jax import lax
from jax.experimental import pallas as pl
from jax.experimental.pallas import tpu as pltpu
from jax.experimental.pallas import tpu_sc as plsc

VOCAB = 1000000
EMBED_DIM = 64
HIDDEN_DIM = 256
OUTPUT_DIM = 1
BATCH = 4096
HIST = 200

NC = 2
NS = 16
LANES = 16
NW = NC * NS
B_PER_W = BATCH // NW
NIDX = B_PER_W * HIST
NBUF = 3
G0, G1 = 128, HIST - 128
NCHUNK = EMBED_DIM // 32

_PERM = []
for _c in range(NCHUNK):
    _PERM += list(range(_c * 32, (_c + 1) * 32, 2))
    _PERM += list(range(_c * 32 + 1, (_c + 1) * 32, 2))


def _start_gather(table_hbm, idx_v, rows, sem, off):
    pltpu.make_async_copy(
        table_hbm.at[idx_v.at[pl.ds(off, G0)]], rows.at[pl.ds(0, G0)], sem
    ).start()
    pltpu.make_async_copy(
        table_hbm.at[idx_v.at[pl.ds(off + G0, G1)]], rows.at[pl.ds(G0, G1)], sem
    ).start()


def _wait_gather(table_hbm, idx_v, rows, sem, off):
    pltpu.make_async_copy(
        table_hbm.at[idx_v.at[pl.ds(off, G0)]], rows.at[pl.ds(0, G0)], sem
    ).wait()
    pltpu.make_async_copy(
        table_hbm.at[idx_v.at[pl.ds(off + G0, G1)]], rows.at[pl.ds(G0, G1)], sem
    ).wait()


_HIMASK = jnp.int32(-65536)


def _reduce_rows(rows, outb, b_local):
    zero = jnp.zeros((LANES,), jnp.float32)

    def body(i, accs):
        r = i * 4
        accs = list(accs)
        for j in range(4):
            for c in range(NCHUNK):
                w = rows[r + j, pl.ds(c * 32, 32)]
                u = plsc.bitcast(w, jnp.int32)
                lo = plsc.bitcast(lax.shift_left(u, 16), jnp.float32)
                hi = plsc.bitcast(u & _HIMASK, jnp.float32)
                accs[c * 2] = accs[c * 2] + lo
                accs[c * 2 + 1] = accs[c * 2 + 1] + hi
        return tuple(accs)

    accs = lax.fori_loop(0, HIST // 4, body, (zero,) * (2 * NCHUNK))
    for k in range(2 * NCHUNK):
        outb[b_local, pl.ds(k * LANES, LANES)] = accs[k]


def _embed_bag(x_flat, table_bf16p):
    mesh = plsc.VectorSubcoreMesh(core_axis_name="c", subcore_axis_name="s")

    @functools.partial(
        pl.kernel,
        mesh=mesh,
        compiler_params=pltpu.CompilerParams(
            use_tc_tiling_on_sc=False, needs_layout_passes=False
        ),
        out_type=jax.ShapeDtypeStruct((BATCH, 2 * EMBED_DIM), jnp.float32),
        scratch_types=[
            pltpu.VMEM((NIDX,), jnp.int32),
            pltpu.VMEM((HIST, 2 * EMBED_DIM), jnp.bfloat16),
            pltpu.VMEM((HIST, 2 * EMBED_DIM), jnp.bfloat16),
            pltpu.VMEM((HIST, 2 * EMBED_DIM), jnp.bfloat16),
            pltpu.VMEM((B_PER_W, 2 * EMBED_DIM), jnp.float32),
            pltpu.SemaphoreType.DMA,
            pltpu.SemaphoreType.DMA,
            pltpu.SemaphoreType.DMA,
        ],
    )
    def k(x_hbm, table_hbm, out_hbm, idx_v, r0, r1, r2, outb, s0, s1, s2):
        wid = lax.axis_index("s") * NC + lax.axis_index("c")
        base = wid * B_PER_W
        pltpu.sync_copy(x_hbm.at[pl.ds(base * HIST, NIDX)], idx_v)

        def zrow(r, _):
            for c in range(4):
                outb[r, pl.ds(EMBED_DIM + c * LANES, LANES)] = jnp.zeros(
                    (LANES,), jnp.float32
                )
            return 0

        lax.fori_loop(0, B_PER_W, zrow, 0)

        bufs = (r0, r1, r2)
        sems = (s0, s1, s2)
        for j in range(NBUF):
            _start_gather(table_hbm, idx_v, bufs[j], sems[j], j * HIST)

        def outer(g, _):
            for j in range(NBUF):
                b = g * NBUF + j
                off = b * HIST
                _wait_gather(table_hbm, idx_v, bufs[j], sems[j], off)
                _reduce_rows(bufs[j], outb, b)
                _start_gather(table_hbm, idx_v, bufs[j], sems[j], off + NBUF * HIST)
            return 0

        n_main = (B_PER_W - NBUF - 2) // NBUF
        lax.fori_loop(0, n_main, outer, 0)
        for b in range(n_main * NBUF, B_PER_W):
            j = b % NBUF
            off = b * HIST
            _wait_gather(table_hbm, idx_v, bufs[j], sems[j], off)
            _reduce_rows(bufs[j], outb, b)
            nxt = b + NBUF
            if nxt < B_PER_W:
                _start_gather(table_hbm, idx_v, bufs[j], sems[j], nxt * HIST)

        pltpu.sync_copy(outb, out_hbm.at[pl.ds(base, B_PER_W)])

    return k(x_flat, table_bf16p)


def _mlp_body(e_ref, w1_ref, b1_ref, w2_ref, b2_ref, w3_ref, b3_ref, out_ref):
    dn = (((1,), (1,)), ((), ()))
    e = e_ref[...]
    l1 = lax.dot_general(e, w1_ref[...], dn, preferred_element_type=jnp.float32)
    l1 = jnp.maximum(l1 + b1_ref[...], 0.0)
    l2 = lax.dot_general(l1, w2_ref[...], dn, preferred_element_type=jnp.float32)
    l2 = jnp.maximum(l2 + b2_ref[...], 0.0)
    out = lax.dot_general(l2, w3_ref[...], dn, preferred_element_type=jnp.float32)
    out_ref[...] = out + b3_ref[...]


def _mlp(e, W1p, b1, W2, b2, W3, b3):
    W3p = jnp.zeros((128, HIDDEN_DIM), W3.dtype).at[:OUTPUT_DIM].set(W3)
    b3p = jnp.zeros((1, 128), b3.dtype).at[0, :OUTPUT_DIM].set(b3)
    out = pl.pallas_call(
        _mlp_body,
        out_shape=jax.ShapeDtypeStruct((BATCH, 128), jnp.float32),
    )(
        e,
        W1p,
        b1.reshape(1, HIDDEN_DIM),
        W2,
        b2.reshape(1, HIDDEN_DIM),
        W3p,
        b3p,
    )
    return out[:, :OUTPUT_DIM]


def kernel(X, table, W1, b1, W2, b2, W3, b3):
    x_flat = X.reshape(-1).astype(jnp.int32)
    tb = jnp.pad(table.astype(jnp.bfloat16), ((0, 0), (0, EMBED_DIM)))
    e = _embed_bag(x_flat, tb)
    W1p = jnp.zeros((HIDDEN_DIM, 2 * EMBED_DIM), W1.dtype).at[:, :EMBED_DIM].set(
        W1[:, jnp.array(_PERM)]
    )
    return _mlp(e, W1p, b1, W2, b2, W3, b3)


# speedup vs baseline: 1.8839x; 1.8839x over previous
import functools

import jax
import jax.numpy as jnp
from jax import lax
from jax.experimental import pallas as pl
from jax.experimental.pallas import tpu as pltpu
from jax.experimental.pallas import tpu_sc as plsc

VOCAB = 1000000
EMBED_DIM = 64
HIDDEN_DIM = 256
OUTPUT_DIM = 1
BATCH = 4096
HIST = 200

NC = 2
NS = 16
LANES = 16
NW = NC * NS
B_PER_W = BATCH // NW
G0, G1 = 128, HIST - 128
NCOL = EMBED_DIM // LANES


def _start_gather(table_hbm, idx_v, rows, sem, off):
    pltpu.make_async_copy(
        table_hbm.at[idx_v.at[pl.ds(off, G0)]], rows.at[pl.ds(0, G0)], sem
    ).start()
    pltpu.make_async_copy(
        table_hbm.at[idx_v.at[pl.ds(off + G0, G1)]], rows.at[pl.ds(G0, G1)], sem
    ).start()


def _wait_gather(table_hbm, idx_v, rows, sem, off):
    pltpu.make_async_copy(
        table_hbm.at[idx_v.at[pl.ds(off, G0)]], rows.at[pl.ds(0, G0)], sem
    ).wait()
    pltpu.make_async_copy(
        table_hbm.at[idx_v.at[pl.ds(off + G0, G1)]], rows.at[pl.ds(G0, G1)], sem
    ).wait()


def _reduce_rows(rows, outb, b_local):
    zero = jnp.zeros((LANES,), jnp.float32)
    def body(i, accs):
        r = i * 4
        accs = list(accs)
        for j in range(4):
            for c in range(NCOL):
                k = c * 2 + (j & 1)
                accs[k] = accs[k] + rows[r + j, pl.ds(c * LANES, LANES)]
        return tuple(accs)

    accs = lax.fori_loop(0, HIST // 4, body, (zero,) * (2 * NCOL))
    for c in range(NCOL):
        outb[b_local, pl.ds(c * LANES, LANES)] = accs[c * 2] + accs[c * 2 + 1]


def _embed_bag(x_flat, table):
    mesh = plsc.VectorSubcoreMesh(core_axis_name="c", subcore_axis_name="s")

    @functools.partial(
        pl.kernel,
        mesh=mesh,
        compiler_params=pltpu.CompilerParams(use_tc_tiling_on_sc=False),
        out_type=jax.ShapeDtypeStruct((BATCH, EMBED_DIM), jnp.float32),
        scratch_types=[
            pltpu.VMEM((B_PER_W * HIST,), jnp.int32),
            pltpu.VMEM((HIST, EMBED_DIM), jnp.float32),
            pltpu.VMEM((HIST, EMBED_DIM), jnp.float32),
            pltpu.VMEM((B_PER_W, EMBED_DIM), jnp.float32),
            pltpu.SemaphoreType.DMA,
            pltpu.SemaphoreType.DMA,
        ],
    )
    def k(x_hbm, table_hbm, out_hbm, idx_v, rows0, rows1, outb, sem0, sem1):
        wid = lax.axis_index("s") * NC + lax.axis_index("c")
        base = wid * B_PER_W
        pltpu.sync_copy(x_hbm.at[pl.ds(base * HIST, B_PER_W * HIST)], idx_v)

        bufs = (rows0, rows1)
        sems = (sem0, sem1)
        for j in range(2):
            _start_gather(table_hbm, idx_v, bufs[j], sems[j], j * HIST)

        def outer(g, _):
            for j in range(2):
                b = g * 2 + j
                off = b * HIST
                _wait_gather(table_hbm, idx_v, bufs[j], sems[j], off)
                _reduce_rows(bufs[j], outb, b)
                _start_gather(table_hbm, idx_v, bufs[j], sems[j], off + 2 * HIST)
            return 0

        lax.fori_loop(0, B_PER_W // 2 - 1, outer, 0)
        for j in range(2):
            b = B_PER_W - 2 + j
            _wait_gather(table_hbm, idx_v, bufs[j], sems[j], b * HIST)
            _reduce_rows(bufs[j], outb, b)

        pltpu.sync_copy(outb, out_hbm.at[pl.ds(base, B_PER_W)])

    return k(x_flat, table)


def _mlp_body(e_ref, w1_ref, b1_ref, w2_ref, b2_ref, w3_ref, b3_ref, out_ref):
    dn = (((1,), (1,)), ((), ()))
    e = e_ref[...]
    l1 = lax.dot_general(e, w1_ref[...], dn, preferred_element_type=jnp.float32)
    l1 = jnp.maximum(l1 + b1_ref[...], 0.0)
    l2 = lax.dot_general(l1, w2_ref[...], dn, preferred_element_type=jnp.float32)
    l2 = jnp.maximum(l2 + b2_ref[...], 0.0)
    out = lax.dot_general(l2, w3_ref[...], dn, preferred_element_type=jnp.float32)
    out_ref[...] = out + b3_ref[...]


def _mlp(e, W1, b1, W2, b2, W3, b3):
    W3p = jnp.zeros((128, HIDDEN_DIM), W3.dtype).at[:OUTPUT_DIM].set(W3)
    b3p = jnp.zeros((1, 128), b3.dtype).at[0, :OUTPUT_DIM].set(b3)
    out = pl.pallas_call(
        _mlp_body,
        out_shape=jax.ShapeDtypeStruct((BATCH, 128), jnp.float32),
    )(
        e,
        W1,
        b1.reshape(1, HIDDEN_DIM),
        W2,
        b2.reshape(1, HIDDEN_DIM),
        W3p,
        b3p,
    )
    return out[:, :OUTPUT_DIM]


def kernel(X, table, W1, b1, W2, b2, W3, b3):
    x_flat = X.reshape(-1).astype(jnp.int32)
    e = _embed_bag(x_flat, table)
    return _mlp(e, W1, b1, W2, b2, W3, b3)


# speedup vs baseline: 1.9605x; 1.0407x over previous
import functools

import jax
import jax.numpy as jnp
from jax import lax
from jax.experimental import pallas as pl
from jax.experimental.pallas import tpu as pltpu
from jax.experimental.pallas import tpu_sc as plsc

VOCAB = 1000000
EMBED_DIM = 64
HIDDEN_DIM = 256
OUTPUT_DIM = 1
BATCH = 4096
HIST = 200

NC = 2
NS = 16
LANES = 16
NW = NC * NS
B_PER_W = BATCH // NW
G0, G1 = 128, HIST - 128
NCOL = EMBED_DIM // LANES


def _start_gather(table_hbm, idx_v, rows, sem, off):
    pltpu.make_async_copy(
        table_hbm.at[idx_v.at[pl.ds(off, G0)]], rows.at[pl.ds(0, G0)], sem
    ).start()
    pltpu.make_async_copy(
        table_hbm.at[idx_v.at[pl.ds(off + G0, G1)]], rows.at[pl.ds(G0, G1)], sem
    ).start()


def _wait_gather(table_hbm, idx_v, rows, sem, off):
    pltpu.make_async_copy(
        table_hbm.at[idx_v.at[pl.ds(off, G0)]], rows.at[pl.ds(0, G0)], sem
    ).wait()
    pltpu.make_async_copy(
        table_hbm.at[idx_v.at[pl.ds(off + G0, G1)]], rows.at[pl.ds(G0, G1)], sem
    ).wait()


def _reduce_rows(rows, outb, b_local):
    zero = jnp.zeros((LANES,), jnp.float32)
    def body(i, accs):
        r = i * 4
        accs = list(accs)
        for j in range(4):
            for c in range(NCOL):
                k = c * 2 + (j & 1)
                accs[k] = accs[k] + rows[r + j, pl.ds(c * LANES, LANES)]
        return tuple(accs)

    accs = lax.fori_loop(0, HIST // 4, body, (zero,) * (2 * NCOL))
    for c in range(NCOL):
        outb[b_local, pl.ds(c * LANES, LANES)] = accs[c * 2] + accs[c * 2 + 1]


def _embed_bag(x_flat, table):
    mesh = plsc.VectorSubcoreMesh(core_axis_name="c", subcore_axis_name="s")

    @functools.partial(
        pl.kernel,
        mesh=mesh,
        compiler_params=pltpu.CompilerParams(use_tc_tiling_on_sc=False),
        out_type=jax.ShapeDtypeStruct((BATCH, EMBED_DIM), jnp.float32),
        scratch_types=[
            pltpu.VMEM((B_PER_W * HIST,), jnp.int32),
            pltpu.VMEM((HIST, EMBED_DIM), jnp.float32),
            pltpu.VMEM((HIST, EMBED_DIM), jnp.float32),
            pltpu.VMEM((HIST, EMBED_DIM), jnp.float32),
            pltpu.VMEM((B_PER_W, EMBED_DIM), jnp.float32),
            pltpu.SemaphoreType.DMA,
            pltpu.SemaphoreType.DMA,
            pltpu.SemaphoreType.DMA,
        ],
    )
    def k(x_hbm, table_hbm, out_hbm, idx_v, rows0, rows1, rows2, outb, s0, s1, s2):
        wid = lax.axis_index("s") * NC + lax.axis_index("c")
        base = wid * B_PER_W
        pltpu.sync_copy(x_hbm.at[pl.ds(base * HIST, B_PER_W * HIST)], idx_v)

        NBUF = 3
        bufs = (rows0, rows1, rows2)
        sems = (s0, s1, s2)
        for j in range(NBUF):
            _start_gather(table_hbm, idx_v, bufs[j], sems[j], j * HIST)

        def outer(g, _):
            for j in range(NBUF):
                b = g * NBUF + j
                off = b * HIST
                _wait_gather(table_hbm, idx_v, bufs[j], sems[j], off)
                _reduce_rows(bufs[j], outb, b)
                _start_gather(table_hbm, idx_v, bufs[j], sems[j], off + NBUF * HIST)
            return 0

        n_main = (B_PER_W - NBUF - 2) // NBUF
        lax.fori_loop(0, n_main, outer, 0)
        for b in range(n_main * NBUF, B_PER_W):
            j = b % NBUF
            _wait_gather(table_hbm, idx_v, bufs[j], sems[j], b * HIST)
            _reduce_rows(bufs[j], outb, b)
            nxt = b + NBUF
            if nxt < B_PER_W:
                _start_gather(table_hbm, idx_v, bufs[j], sems[j], nxt * HIST)

        pltpu.sync_copy(outb, out_hbm.at[pl.ds(base, B_PER_W)])

    return k(x_flat, table)


def _mlp_body(e_ref, w1_ref, b1_ref, w2_ref, b2_ref, w3_ref, b3_ref, out_ref):
    dn = (((1,), (1,)), ((), ()))
    e = e_ref[...]
    l1 = lax.dot_general(e, w1_ref[...], dn, preferred_element_type=jnp.float32)
    l1 = jnp.maximum(l1 + b1_ref[...], 0.0)
    l2 = lax.dot_general(l1, w2_ref[...], dn, preferred_element_type=jnp.float32)
    l2 = jnp.maximum(l2 + b2_ref[...], 0.0)
    out = lax.dot_general(l2, w3_ref[...], dn, preferred_element_type=jnp.float32)
    out_ref[...] = out + b3_ref[...]


def _mlp(e, W1, b1, W2, b2, W3, b3):
    W3p = jnp.zeros((128, HIDDEN_DIM), W3.dtype).at[:OUTPUT_DIM].set(W3)
    b3p = jnp.zeros((1, 128), b3.dtype).at[0, :OUTPUT_DIM].set(b3)
    out = pl.pallas_call(
        _mlp_body,
        out_shape=jax.ShapeDtypeStruct((BATCH, 128), jnp.float32),
    )(
        e,
        W1,
        b1.reshape(1, HIDDEN_DIM),
        W2,
        b2.reshape(1, HIDDEN_DIM),
        W3p,
        b3p,
    )
    return out[:, :OUTPUT_DIM]


def kernel(X, table, W1, b1, W2, b2, W3, b3):
    x_flat = X.reshape(-1).astype(jnp.int32)
    e = _embed_bag(x_flat, table)
    return _mlp(e, W1, b1, W2, b2, W3, b3)
